# baseline (speedup 1.0000x reference)
import dataclasses
import functools

import jax
import jax.numpy as jnp
from jax import lax
from jax.experimental import pallas as pl
from jax.experimental.pallas import tpu as pltpu
from jax.experimental.pallas import tpu_sc as plsc

_B = 4096
_D_IN = 1024
_D_H = 1024
_SW = 16384

_NC = 2
_NS = 16
_NW = _NC * _NS
_BPW = _B // _NW

_BM = 2048
_RPB = _BM // _BPW


def _sc_gather(table, index):
  mesh = plsc.VectorSubcoreMesh(core_axis_name="c", subcore_axis_name="s")
  cp = pltpu.CompilerParams()
  if "needs_layout_passes" in pltpu.CompilerParams.__dataclass_fields__:
    cp = dataclasses.replace(cp, needs_layout_passes=False)

  @functools.partial(
      pl.kernel,
      mesh=mesh,
      compiler_params=cp,
      out_type=jax.ShapeDtypeStruct((_NW, _BPW), jnp.float32),
      scratch_types=[
          pltpu.VMEM((_BPW, 1), jnp.int32),
          pltpu.VMEM((_BPW,), jnp.int32),
          pltpu.VMEM((_BPW,), jnp.float32),
          pltpu.VMEM((1, _BPW), jnp.float32),
          pltpu.SemaphoreType.DMA,
      ],
  )
  def k(table_hbm, idx_hbm, out_hbm, idx_v2, idx_lin, vals_v, out_v, sem):
    wid = lax.axis_index("s") * _NC + lax.axis_index("c")
    base = wid * _BPW
    pltpu.sync_copy(idx_hbm.at[pl.ds(base, _BPW)], idx_v2)
    k16 = lax.iota(jnp.int32, 16)
    z16 = jnp.zeros((16,), jnp.int32)

    @pl.loop(0, _BPW, step=16)
    def _(j):
      idx_lin[pl.ds(j, 16)] = plsc.load_gather(idx_v2, [j + k16, z16])

    pltpu.async_copy(table_hbm.at[idx_lin], vals_v, sem).wait()
    ov = out_v.at[0]

    @pl.loop(0, _BPW, step=16)
    def _(j):
      ov[pl.ds(j, 16)] = vals_v[pl.ds(j, 16)]

    pltpu.sync_copy(out_v, out_hbm.at[pl.ds(wid, 1)])

  return k(table, index)


def _tc_body(x_ref, y_ref, sw_ref, w1_ref, b1_ref, w2_ref, b2_ref, out_ref):
  xb = x_ref[...].astype(jnp.bfloat16)
  w1b = w1_ref[...].astype(jnp.bfloat16)
  h = jnp.dot(xb, w1b, preferred_element_type=jnp.float32) + b1_ref[...]
  hb = jnp.maximum(h, 0.0).astype(jnp.bfloat16)
  w2b = w2_ref[...].astype(jnp.bfloat16)
  pred = jnp.dot(hb, w2b, preferred_element_type=jnp.float32)
  e = pred + b2_ref[...] - y_ref[...]
  e2 = e * e
  i = pl.program_id(0)
  parts = [
      jnp.dot(sw_ref[pl.ds(i * _RPB + r, 1), :],
              e2[r * _BPW : (r + 1) * _BPW, :],
              preferred_element_type=jnp.float32)
      for r in range(_RPB)
  ]
  while len(parts) > 1:
    parts = [parts[i2] + parts[i2 + 1] for i2 in range(0, len(parts), 2)]
  partial = parts[0] * (1.0 / _B)

  @pl.when(i == 0)
  def _():
    out_ref[...] = jnp.zeros_like(out_ref)

  out_ref[...] += partial


def _tc_loss(x, y, swg, w1, b1, w2, b2):
  out = pl.pallas_call(
      _tc_body,
      grid=(_B // _BM,),
      in_specs=[
          pl.BlockSpec((_BM, _D_IN), lambda i: (i, 0)),
          pl.BlockSpec((_BM, 1), lambda i: (i, 0)),
          pl.BlockSpec((_NW, _BPW), lambda i: (0, 0)),
          pl.BlockSpec((_D_IN, _D_H), lambda i: (0, 0)),
          pl.BlockSpec((1, _D_H), lambda i: (0, 0)),
          pl.BlockSpec((_D_H, 1), lambda i: (0, 0)),
          pl.BlockSpec((1, 1), lambda i: (0, 0)),
      ],
      out_specs=pl.BlockSpec((1, 1), lambda i: (0, 0)),
      out_shape=jax.ShapeDtypeStruct((1, 1), jnp.float32),
      compiler_params=pltpu.CompilerParams(
          dimension_semantics=("arbitrary",),
      ),
  )(x, y, swg, w1, b1, w2, b2)
  return out.reshape(())


def kernel(x, y, index, W1, b1, W2, b2, sample_weights):
  swg = _sc_gather(sample_weights, index)
  return _tc_loss(
      x, y, swg, W1,
      b1.reshape(1, _D_H),
      W2,
      b2.reshape(1, 1),
  )


# speedup vs baseline: 1.0231x; 1.0231x over previous
import dataclasses
import functools

import jax
import jax.numpy as jnp
from jax import lax
from jax.experimental import pallas as pl
from jax.experimental.pallas import tpu as pltpu
from jax.experimental.pallas import tpu_sc as plsc

_B = 4096
_D_IN = 1024
_D_H = 1024
_SW = 16384

_NC = 2
_NS = 16
_NW = _NC * _NS
_BPW = _B // _NW

_BM = 1024
_RPB = _BM // _BPW


def _sc_gather(table, index):
  mesh = plsc.VectorSubcoreMesh(core_axis_name="c", subcore_axis_name="s")
  cp = pltpu.CompilerParams()
  if "needs_layout_passes" in pltpu.CompilerParams.__dataclass_fields__:
    cp = dataclasses.replace(cp, needs_layout_passes=False)

  @functools.partial(
      pl.kernel,
      mesh=mesh,
      compiler_params=cp,
      out_type=jax.ShapeDtypeStruct((_NW, _BPW), jnp.float32),
      scratch_types=[
          pltpu.VMEM((_BPW, 1), jnp.int32),
          pltpu.VMEM((_BPW,), jnp.int32),
          pltpu.VMEM((1, _BPW), jnp.float32),
          pltpu.SemaphoreType.DMA,
      ],
  )
  def k(table_hbm, idx_hbm, out_hbm, idx_v2, idx_lin, out_v, sem):
    wid = lax.axis_index("s") * _NC + lax.axis_index("c")
    base = wid * _BPW
    pltpu.sync_copy(idx_hbm.at[pl.ds(base, _BPW)], idx_v2)
    k16 = lax.iota(jnp.int32, 16)
    z16 = jnp.zeros((16,), jnp.int32)

    @pl.loop(0, _BPW, step=16)
    def _(j):
      idx_lin[pl.ds(j, 16)] = plsc.load_gather(idx_v2, [j + k16, z16])

    pltpu.async_copy(table_hbm.at[idx_lin], out_v.at[0], sem).wait()
    pltpu.sync_copy(out_v, out_hbm.at[pl.ds(wid, 1)])

  return k(table, index)


def _tc_body(x_ref, y_ref, sw_ref, w1_ref, b1_ref, w2_ref, b2_ref, out_ref):
  xb = x_ref[...].astype(jnp.bfloat16)
  w1b = w1_ref[...].astype(jnp.bfloat16)
  h = jnp.dot(xb, w1b, preferred_element_type=jnp.float32) + b1_ref[...]
  hb = jnp.maximum(h, 0.0).astype(jnp.bfloat16)
  w2b = w2_ref[...].astype(jnp.bfloat16)
  pred = jnp.dot(hb, w2b, preferred_element_type=jnp.float32)
  e = pred + b2_ref[...] - y_ref[...]
  e2 = e * e
  i = pl.program_id(0)
  parts = [
      jnp.dot(sw_ref[pl.ds(i * _RPB + r, 1), :],
              e2[r * _BPW : (r + 1) * _BPW, :],
              preferred_element_type=jnp.float32)
      for r in range(_RPB)
  ]
  while len(parts) > 1:
    parts = [parts[i2] + parts[i2 + 1] for i2 in range(0, len(parts), 2)]
  partial = parts[0] * (1.0 / _B)

  @pl.when(i == 0)
  def _():
    out_ref[...] = jnp.zeros_like(out_ref)

  out_ref[...] += partial


def _tc_loss(x, y, swg, w1, b1, w2, b2):
  out = pl.pallas_call(
      _tc_body,
      grid=(_B // _BM,),
      in_specs=[
          pl.BlockSpec((_BM, _D_IN), lambda i: (i, 0)),
          pl.BlockSpec((_BM, 1), lambda i: (i, 0)),
          pl.BlockSpec((_NW, _BPW), lambda i: (0, 0)),
          pl.BlockSpec((_D_IN, _D_H), lambda i: (0, 0)),
          pl.BlockSpec((1, _D_H), lambda i: (0, 0)),
          pl.BlockSpec((_D_H, 1), lambda i: (0, 0)),
          pl.BlockSpec((1, 1), lambda i: (0, 0)),
      ],
      out_specs=pl.BlockSpec((1, 1), lambda i: (0, 0)),
      out_shape=jax.ShapeDtypeStruct((1, 1), jnp.float32),
      compiler_params=pltpu.CompilerParams(
          dimension_semantics=("arbitrary",),
      ),
  )(x, y, swg, w1, b1, w2, b2)
  return out.reshape(())


def kernel(x, y, index, W1, b1, W2, b2, sample_weights):
  swg = _sc_gather(sample_weights, index)
  return _tc_loss(
      x, y, swg, W1,
      b1.reshape(1, _D_H),
      W2,
      b2.reshape(1, 1),
  )
